# baseline (speedup 1.0000x reference)
import functools

import jax
import jax.numpy as jnp
from jax import lax
from jax.experimental import pallas as pl
from jax.experimental.pallas import tpu as pltpu
from jax.experimental.pallas import tpu_sc as plsc

NCLS = 50
BATCH = 64
NPTS = 4096
BPB = 8
HALF = BATCH // 2
TC_GRID = HALF // BPB

NWORK = 32
ROWS_PER_W = HALF // NWORK
HWORDS = 192


def _argmax_body(logits_ref, pred_ref):
    best = logits_ref[0]
    idx = jnp.zeros((BPB, NPTS), jnp.int32)
    for c in range(1, NCLS):
        v = logits_ref[c]
        gt = v > best
        best = jnp.where(gt, v, best)
        idx = jnp.where(gt, c, idx)
    pred_ref[...] = idx


def _tc_argmax(logits_cmajor, h):
    return pl.pallas_call(
        _argmax_body,
        grid=(TC_GRID,),
        in_specs=[
            pl.BlockSpec((NCLS, BPB, NPTS), lambda i, h=h: (0, i + TC_GRID * h, 0))
        ],
        out_specs=pl.BlockSpec((BPB, NPTS), lambda i: (i, 0)),
        out_shape=jax.ShapeDtypeStruct((HALF, NPTS), jnp.int32),
    )(logits_cmajor)


def _make_sc_hist(h):
    @functools.partial(
        pl.kernel,
        out_type=jax.ShapeDtypeStruct((NWORK, HWORDS), jnp.float32),
        mesh=plsc.VectorSubcoreMesh(core_axis_name="c", subcore_axis_name="s"),
        compiler_params=pltpu.CompilerParams(needs_layout_passes=False),
        scratch_types=[
            pltpu.VMEM((ROWS_PER_W, NPTS), jnp.int32),
            pltpu.VMEM((ROWS_PER_W, NPTS), jnp.int32),
            pltpu.VMEM((16 * HWORDS,), jnp.float32),
            pltpu.VMEM((HWORDS,), jnp.float32),
        ],
    )
    def _sc_hist(pred_hbm, lab_hbm, out_hbm, pv, lv, lhist, hist):
        cid = lax.axis_index("c")
        sid = lax.axis_index("s")
        w = sid * 2 + cid
        pltpu.sync_copy(pred_hbm.at[pl.ds(w * ROWS_PER_W, ROWS_PER_W)], pv)
        pltpu.sync_copy(
            lab_hbm.at[pl.ds(h * HALF + w * ROWS_PER_W, ROWS_PER_W)], lv
        )

        zeros16 = jnp.zeros((16,), jnp.float32)
        for j in range(16 * HWORDS // 16):
            lhist[pl.ds(j * 16, 16)] = zeros16

        ones = jnp.ones((16,), jnp.float32)
        lbase = lax.iota(jnp.int32, 16) * HWORDS

        for r in range(ROWS_PER_W):
            def body(k, carry):
                off = k * 16
                p = pv[r, pl.ds(off, 16)]
                l = lv[r, pl.ds(off, 16)]
                mval = jnp.where(p == l, 1.0, 0.0).astype(jnp.float32)
                ip = lbase + p
                plsc.addupdate_scatter(lhist, [lbase + (l + 128)], ones)
                plsc.addupdate_scatter(lhist, [ip], mval)
                plsc.addupdate_scatter(lhist, [ip + 64], ones)
                return carry
            lax.fori_loop(0, NPTS // 16, body, 0)

        for j in range(HWORDS // 16):
            acc = zeros16
            for t in range(16):
                acc = acc + lhist[pl.ds(t * HWORDS + j * 16, 16)]
            hist[pl.ds(j * 16, 16)] = acc

        pltpu.sync_copy(hist, out_hbm.at[w])

    return _sc_hist


_SC_HIST = (_make_sc_hist(0), _make_sc_hist(1))


@jax.jit
def kernel(seg_logits, seg_labels):
    lt = jnp.transpose(seg_logits, (1, 0, 2))
    pred0 = _tc_argmax(lt, 0)
    part0 = _SC_HIST[0](pred0, seg_labels)
    pred1 = _tc_argmax(lt, 1)
    part1 = _SC_HIST[1](pred1, seg_labels)
    res = jnp.sum(part0, axis=0) + jnp.sum(part1, axis=0)
    inter = res[0:NCLS]
    union = res[64:64 + NCLS] + res[128:128 + NCLS] - inter
    return inter, union
